# baseline (speedup 1.0000x reference)
import functools

import jax
import jax.numpy as jnp
from jax.experimental import pallas as pl
from jax.experimental.pallas import tpu as pltpu

_BN = 1024


def _gauss_body(std_row_ref, mu_ref, x_hbm, out_hbm,
                wbuf, msqh_buf, xbuf, obuf, in_sem, out_sem, *, steps):
    base = pl.program_id(0) * (steps * _BN)

    inv_row = 1.0 / std_row_ref[...]
    mu0 = mu_ref[...]
    wbuf[...] = mu0 * inv_row
    msqh_buf[...] = 0.5 * jax.lax.dot_general(
        inv_row, mu0 * mu0,
        dimension_numbers=(((1,), (1,)), ((), ())),
        preferred_element_type=jnp.float32)

    def start_in(slot, t):
        pltpu.make_async_copy(
            x_hbm.at[pl.ds(base + t * _BN, _BN), :],
            xbuf.at[slot], in_sem.at[slot]).start()

    def wait_in(slot):
        pltpu.make_async_copy(
            x_hbm.at[pl.ds(0, _BN), :],
            xbuf.at[slot], in_sem.at[slot]).wait()

    def start_out(slot, t):
        pltpu.make_async_copy(
            obuf.at[slot],
            out_hbm.at[pl.ds(base + t * _BN, _BN), :],
            out_sem.at[slot]).start()

    def wait_out(slot):
        pltpu.make_async_copy(
            obuf.at[slot],
            out_hbm.at[pl.ds(0, _BN), :],
            out_sem.at[slot]).wait()

    start_in(0, 0)
    start_in(1, 1)
    msqh = msqh_buf[...]

    def body(t, _):
        cur = jax.lax.rem(t, 2)
        wait_in(cur)

        @pl.when(t >= 2)
        def _():
            wait_out(cur)

        xb = xbuf[cur]
        xsq_half = 0.5 * jnp.sum(xb * xb * inv_row, axis=1, keepdims=True)
        cross = jax.lax.dot_general(
            xb, wbuf[...],
            dimension_numbers=(((1,), (1,)), ((), ())),
            preferred_element_type=jnp.float32)
        obuf[cur] = jnp.exp(cross - xsq_half - msqh)
        start_out(cur, t)

        @pl.when(t + 2 < steps)
        def _():
            start_in(cur, t + 2)
        return ()

    jax.lax.fori_loop(0, steps, body, ())
    wait_out(0)
    wait_out(1)


def kernel(x, mu, std):
    n, d = x.shape
    k = mu.shape[0]
    mu0 = mu[:, 0, :]
    std_row = std.reshape(1, d)
    return pl.pallas_call(
        functools.partial(_gauss_body, steps=n // (2 * _BN)),
        grid=(2,),
        in_specs=[
            pl.BlockSpec((1, d), lambda p: (0, 0)),
            pl.BlockSpec((k, d), lambda p: (0, 0)),
            pl.BlockSpec(memory_space=pl.ANY),
        ],
        out_specs=pl.BlockSpec(memory_space=pl.ANY),
        out_shape=jax.ShapeDtypeStruct((n, k), jnp.float32),
        scratch_shapes=[
            pltpu.VMEM((k, d), jnp.float32),
            pltpu.VMEM((1, k), jnp.float32),
            pltpu.VMEM((2, _BN, d), jnp.float32),
            pltpu.VMEM((2, _BN, k), jnp.float32),
            pltpu.SemaphoreType.DMA((2,)),
            pltpu.SemaphoreType.DMA((2,)),
        ],
        compiler_params=pltpu.CompilerParams(
            dimension_semantics=("parallel",),
            vmem_limit_bytes=60 * 1024 * 1024,
        ),
    )(std_row, mu0, x)


# speedup vs baseline: 1.0116x; 1.0116x over previous
import functools

import jax
import jax.numpy as jnp
from jax.experimental import pallas as pl
from jax.experimental.pallas import tpu as pltpu

_BN = 1024


def _gauss_body(std_row_ref, mu_ref, x_hbm, out_hbm,
                wbuf, msqh_buf, xbuf, obuf, in_sem, out_sem, *, steps):
    base = pl.program_id(0) * (steps * _BN)

    inv_row = 1.0 / std_row_ref[...]
    mu0 = mu_ref[...]
    wbuf[...] = mu0 * inv_row
    msqh_buf[...] = 0.5 * jax.lax.dot_general(
        inv_row, mu0 * mu0,
        dimension_numbers=(((1,), (1,)), ((), ())),
        preferred_element_type=jnp.float32)

    def start_in(slot, t):
        pltpu.make_async_copy(
            x_hbm.at[pl.ds(base + t * _BN, _BN), :],
            xbuf.at[slot], in_sem.at[slot]).start()

    def wait_in(slot):
        pltpu.make_async_copy(
            x_hbm.at[pl.ds(0, _BN), :],
            xbuf.at[slot], in_sem.at[slot]).wait()

    def start_out(slot, t):
        pltpu.make_async_copy(
            obuf.at[slot],
            out_hbm.at[pl.ds(base + t * _BN, _BN), :],
            out_sem.at[slot]).start()

    def wait_out(slot):
        pltpu.make_async_copy(
            obuf.at[slot],
            out_hbm.at[pl.ds(0, _BN), :],
            out_sem.at[slot]).wait()

    start_in(0, 0)
    start_in(1, 1)
    msqh = msqh_buf[...]

    def body(t, _):
        cur = jax.lax.rem(t, 2)
        ocur = jax.lax.rem(t, 3)
        wait_in(cur)

        @pl.when(t >= 3)
        def _():
            wait_out(ocur)

        xb = xbuf[cur]
        xsq_half = 0.5 * jnp.sum(xb * xb * inv_row, axis=1, keepdims=True)
        cross = jax.lax.dot_general(
            xb, wbuf[...],
            dimension_numbers=(((1,), (1,)), ((), ())),
            preferred_element_type=jnp.float32)
        obuf[ocur] = jnp.exp(cross - xsq_half - msqh)
        start_out(ocur, t)

        @pl.when(t + 2 < steps)
        def _():
            start_in(cur, t + 2)
        return ()

    jax.lax.fori_loop(0, steps, body, ())
    wait_out(jax.lax.rem(steps - 3, 3))
    wait_out(jax.lax.rem(steps - 2, 3))
    wait_out(jax.lax.rem(steps - 1, 3))


def kernel(x, mu, std):
    n, d = x.shape
    k = mu.shape[0]
    mu0 = mu[:, 0, :]
    std_row = std.reshape(1, d)
    return pl.pallas_call(
        functools.partial(_gauss_body, steps=n // (2 * _BN)),
        grid=(2,),
        in_specs=[
            pl.BlockSpec((1, d), lambda p: (0, 0)),
            pl.BlockSpec((k, d), lambda p: (0, 0)),
            pl.BlockSpec(memory_space=pl.ANY),
        ],
        out_specs=pl.BlockSpec(memory_space=pl.ANY),
        out_shape=jax.ShapeDtypeStruct((n, k), jnp.float32),
        scratch_shapes=[
            pltpu.VMEM((k, d), jnp.float32),
            pltpu.VMEM((1, k), jnp.float32),
            pltpu.VMEM((2, _BN, d), jnp.float32),
            pltpu.VMEM((3, _BN, k), jnp.float32),
            pltpu.SemaphoreType.DMA((2,)),
            pltpu.SemaphoreType.DMA((3,)),
        ],
        compiler_params=pltpu.CompilerParams(
            dimension_semantics=("parallel",),
            vmem_limit_bytes=60 * 1024 * 1024,
        ),
    )(std_row, mu0, x)


# speedup vs baseline: 1.0389x; 1.0269x over previous
import jax
import jax.numpy as jnp
from jax.experimental import pallas as pl
from jax.experimental.pallas import tpu as pltpu

_BN = 1024


def _gauss_body(std_row_ref, mu_ref, x_ref, out_ref):
    inv_row = 1.0 / std_row_ref[...]
    mu0 = mu_ref[...]
    muw = mu0 * inv_row
    msq_half = 0.5 * jax.lax.dot_general(
        inv_row, mu0 * mu0,
        dimension_numbers=(((1,), (1,)), ((), ())),
        preferred_element_type=jnp.float32)
    xb = x_ref[...]
    xsq_half = 0.5 * jnp.sum(xb * xb * inv_row, axis=1, keepdims=True)
    cross = jax.lax.dot_general(
        xb, muw,
        dimension_numbers=(((1,), (1,)), ((), ())),
        preferred_element_type=jnp.float32)
    out_ref[...] = jnp.exp(cross - xsq_half - msq_half)


def kernel(x, mu, std):
    n, d = x.shape
    k = mu.shape[0]
    mu0 = mu[:, 0, :]
    std_row = std.reshape(1, d)
    return pl.pallas_call(
        _gauss_body,
        grid=(n // _BN,),
        in_specs=[
            pl.BlockSpec((1, d), lambda i: (0, 0)),
            pl.BlockSpec((k, d), lambda i: (0, 0)),
            pl.BlockSpec((_BN, d), lambda i: (i, 0)),
        ],
        out_specs=pl.BlockSpec((_BN, k), lambda i: (i, 0)),
        out_shape=jax.ShapeDtypeStruct((n, k), jnp.float32),
        compiler_params=pltpu.CompilerParams(
            dimension_semantics=("parallel",),
            vmem_limit_bytes=60 * 1024 * 1024,
        ),
    )(std_row, mu0, x)
